# Initial kernel scaffold; baseline (speedup 1.0000x reference)
import jax
import jax.numpy as jnp
from jax.experimental import pallas as pl


def kernel(input_seqs, input_lens, emb, Kg0, bg0, Kc0, bc0, Kg1, bg1, Kc1, bc1, Kg2, bg2, Kc2, bc2, W, b):
    raise NotImplementedError("write your pallas kernel here")


# speedup vs baseline: 1.8573x; 1.8573x over previous
import jax
import jax.numpy as jnp
from jax.experimental import pallas as pl
from jax.experimental.pallas import tpu as pltpu

VOCAB = 40
EMB = 32
S0, S1, S2 = 512, 1024, 2048
LATENT = 512
B, T = 256, 128
BH = B // 2
OHV = 128


def _gru_body(seq_ref, len_ref, bg0r, bc0r, bg1r, bc1r, bg2r, bc2r, bfr,
              e0a, gh0a, ch0a, x1a, gh1a, ch1a, x2a, gh2a, ch2a, w0a, w1a, w2a,
              out_ref,
              e0, gh0, ch0, x1, gh1, ch1, x2, gh2, ch2, w0, w1, w2,
              h0, h1, h2, sems):
    srcs = (e0a, gh0a, ch0a, x1a, gh1a, ch1a, x2a, gh2a, ch2a, w0a, w1a, w2a)
    dsts = (e0, gh0, ch0, x1, gh1, ch1, x2, gh2, ch2, w0, w1, w2)
    for i, (s, d) in enumerate(zip(srcs, dsts)):
        pltpu.make_async_copy(s, d, sems.at[i]).start()
    for i, (s, d) in enumerate(zip(srcs, dsts)):
        pltpu.make_async_copy(s, d, sems.at[i]).wait()

    h0[...] = jnp.zeros((BH, S0), jnp.float32)
    h1[...] = jnp.zeros((BH, S1), jnp.float32)
    h2[...] = jnp.zeros((BH, S2), jnp.float32)

    viota = jax.lax.broadcasted_iota(jnp.int32, (OHV, BH), 0)

    def cell(xc, h_ref, ghw, chw, bg, bc, outs):
        h = h_ref[...]
        hb = h.astype(jnp.bfloat16)
        g = jax.nn.sigmoid(
            xc[:, : 2 * outs]
            + jnp.dot(hb, ghw[...], preferred_element_type=jnp.float32)
            + bg[...])
        r = g[:, :outs]
        z = g[:, outs:]
        c = jnp.tanh(
            xc[:, 2 * outs:]
            + jnp.dot((r * h).astype(jnp.bfloat16), chw[...],
                      preferred_element_type=jnp.float32)
            + bc[...])
        return z * h + (1.0 - z) * c

    def step(t, _):
        ids = seq_ref[t]
        ohT = jnp.where(ids == viota, 1.0, 0.0)
        oh = jnp.transpose(ohT).astype(jnp.bfloat16)
        xc0 = jnp.dot(oh, e0[...], preferred_element_type=jnp.float32)
        n0 = cell(xc0, h0, gh0, ch0, bg0r, bc0r, S0)
        xc1 = jnp.dot(n0.astype(jnp.bfloat16), x1[...], preferred_element_type=jnp.float32)
        n1 = cell(xc1, h1, gh1, ch1, bg1r, bc1r, S1)
        xc2 = jnp.dot(n1.astype(jnp.bfloat16), x2[...], preferred_element_type=jnp.float32)
        n2 = cell(xc2, h2, gh2, ch2, bg2r, bc2r, S2)
        m = len_ref[...] > t
        h0[...] = jnp.where(m, n0, h0[...])
        h1[...] = jnp.where(m, n1, h1[...])
        h2[...] = jnp.where(m, n2, h2[...])
        return 0

    jax.lax.fori_loop(0, T, step, 0)

    p = (jnp.dot(h0[...].astype(jnp.bfloat16), w0[...], preferred_element_type=jnp.float32)
         + jnp.dot(h1[...].astype(jnp.bfloat16), w1[...], preferred_element_type=jnp.float32)
         + jnp.dot(h2[...].astype(jnp.bfloat16), w2[...], preferred_element_type=jnp.float32)
         + bfr[...])
    out_ref[...] = jnp.tanh(p)


def kernel(input_seqs, input_lens, emb, Kg0, bg0, Kc0, bc0, Kg1, bg1, Kc1, bc1,
           Kg2, bg2, Kc2, bc2, W, b):
    f32 = jnp.float32
    bf16 = jnp.bfloat16

    kx0 = jnp.concatenate([Kg0[:EMB], Kc0[:EMB]], axis=1)
    e0 = jnp.zeros((OHV, 3 * S0), f32).at[:VOCAB].set(emb.astype(f32) @ kx0)
    e0 = e0.astype(bf16)
    gh0w = Kg0[EMB:].astype(bf16)
    ch0w = Kc0[EMB:].astype(bf16)
    x1w = jnp.concatenate([Kg1[:S0], Kc1[:S0]], axis=1).astype(bf16)
    gh1w = Kg1[S0:].astype(bf16)
    ch1w = Kc1[S0:].astype(bf16)
    x2w = jnp.concatenate([Kg2[:S1], Kc2[:S1]], axis=1).astype(bf16)
    gh2w = Kg2[S1:].astype(bf16)
    ch2w = Kc2[S1:].astype(bf16)
    wt = jnp.transpose(W)
    w0t = wt[:S0].astype(bf16)
    w1t = wt[S0:S0 + S1].astype(bf16)
    w2t = wt[S0 + S1:].astype(bf16)

    seqs = jnp.transpose(input_seqs).reshape(T, 1, B)
    lens = input_lens.reshape(B, 1)
    bg0r = bg0.reshape(1, -1)
    bc0r = bc0.reshape(1, -1)
    bg1r = bg1.reshape(1, -1)
    bc1r = bc1.reshape(1, -1)
    bg2r = bg2.reshape(1, -1)
    bc2r = bc2.reshape(1, -1)
    bfr = b.reshape(1, -1)

    full = lambda shape: pl.BlockSpec(shape, lambda i: tuple(0 for _ in shape))
    anyspec = pl.BlockSpec(memory_space=pl.ANY)

    out = pl.pallas_call(
        _gru_body,
        grid=(2,),
        in_specs=[
            pl.BlockSpec((T, 1, BH), lambda i: (0, 0, i)),
            pl.BlockSpec((BH, 1), lambda i: (i, 0)),
            full((1, 2 * S0)), full((1, S0)),
            full((1, 2 * S1)), full((1, S1)),
            full((1, 2 * S2)), full((1, S2)),
            full((1, LATENT)),
        ] + [anyspec] * 12,
        out_specs=pl.BlockSpec((BH, LATENT), lambda i: (i, 0)),
        out_shape=jax.ShapeDtypeStruct((B, LATENT), f32),
        scratch_shapes=[
            pltpu.VMEM((OHV, 3 * S0), bf16),
            pltpu.VMEM((S0, 2 * S0), bf16),
            pltpu.VMEM((S0, S0), bf16),
            pltpu.VMEM((S0, 3 * S1), bf16),
            pltpu.VMEM((S1, 2 * S1), bf16),
            pltpu.VMEM((S1, S1), bf16),
            pltpu.VMEM((S1, 3 * S2), bf16),
            pltpu.VMEM((S2, 2 * S2), bf16),
            pltpu.VMEM((S2, S2), bf16),
            pltpu.VMEM((S0, LATENT), bf16),
            pltpu.VMEM((S1, LATENT), bf16),
            pltpu.VMEM((S2, LATENT), bf16),
            pltpu.VMEM((BH, S0), f32),
            pltpu.VMEM((BH, S1), f32),
            pltpu.VMEM((BH, S2), f32),
            pltpu.SemaphoreType.DMA((12,)),
        ],
        compiler_params=pltpu.CompilerParams(
            dimension_semantics=("parallel",),
            vmem_limit_bytes=64 * 1024 * 1024,
        ),
        name="mini_cddd_gru",
    )(seqs, lens, bg0r, bc0r, bg1r, bc1r, bg2r, bc2r, bfr,
      e0, gh0w, ch0w, x1w, gh1w, ch1w, x2w, gh2w, ch2w, w0t, w1t, w2t)
    return out


# speedup vs baseline: 1.9386x; 1.0438x over previous
import jax
import jax.numpy as jnp
from jax.experimental import pallas as pl
from jax.experimental.pallas import tpu as pltpu

VOCAB = 40
EMB = 32
S0, S1, S2 = 512, 1024, 2048
LATENT = 512
B, T = 256, 128
BH = B
OHV = 128


def _gru_body(seq_ref, len_ref, bg0r, bc0r, bg1r, bc1r, bg2r, bc2r, bfr,
              e0a, gh0a, ch0a, x1a, gh1a, ch1a, x2a, gh2a, ch2a, w0a, w1a, w2a,
              out_ref,
              e0, gh0, ch0, x1, gh1, ch1, x2, gh2, ch2, w0, w1, w2,
              h0, h1, h2, sems):
    srcs = (e0a, gh0a, ch0a, x1a, gh1a, ch1a, x2a, gh2a, ch2a, w0a, w1a, w2a)
    dsts = (e0, gh0, ch0, x1, gh1, ch1, x2, gh2, ch2, w0, w1, w2)
    for i, (s, d) in enumerate(zip(srcs, dsts)):
        pltpu.make_async_copy(s, d, sems.at[i]).start()
    for i, (s, d) in enumerate(zip(srcs, dsts)):
        pltpu.make_async_copy(s, d, sems.at[i]).wait()

    h0[...] = jnp.zeros((BH, S0), jnp.float32)
    h1[...] = jnp.zeros((BH, S1), jnp.float32)
    h2[...] = jnp.zeros((BH, S2), jnp.float32)

    viota = jax.lax.broadcasted_iota(jnp.int32, (OHV, BH), 0)

    def cell(xc, h_ref, ghw, chw, bg, bc, outs):
        h = h_ref[...]
        hb = h.astype(jnp.bfloat16)
        g = jax.nn.sigmoid(
            xc[:, : 2 * outs]
            + jnp.dot(hb, ghw[...], preferred_element_type=jnp.float32)
            + bg[...])
        r = g[:, :outs]
        z = g[:, outs:]
        c = jnp.tanh(
            xc[:, 2 * outs:]
            + jnp.dot((r * h).astype(jnp.bfloat16), chw[...],
                      preferred_element_type=jnp.float32)
            + bc[...])
        return z * h + (1.0 - z) * c

    def step(t, _):
        ids = seq_ref[t]
        ohT = jnp.where(ids == viota, 1.0, 0.0)
        oh = jnp.transpose(ohT).astype(jnp.bfloat16)
        xc0 = jnp.dot(oh, e0[...], preferred_element_type=jnp.float32)
        n0 = cell(xc0, h0, gh0, ch0, bg0r, bc0r, S0)
        xc1 = jnp.dot(n0.astype(jnp.bfloat16), x1[...], preferred_element_type=jnp.float32)
        n1 = cell(xc1, h1, gh1, ch1, bg1r, bc1r, S1)
        xc2 = jnp.dot(n1.astype(jnp.bfloat16), x2[...], preferred_element_type=jnp.float32)
        n2 = cell(xc2, h2, gh2, ch2, bg2r, bc2r, S2)
        m = len_ref[...] > t
        h0[...] = jnp.where(m, n0, h0[...])
        h1[...] = jnp.where(m, n1, h1[...])
        h2[...] = jnp.where(m, n2, h2[...])
        return 0

    jax.lax.fori_loop(0, T, step, 0)

    p = (jnp.dot(h0[...].astype(jnp.bfloat16), w0[...], preferred_element_type=jnp.float32)
         + jnp.dot(h1[...].astype(jnp.bfloat16), w1[...], preferred_element_type=jnp.float32)
         + jnp.dot(h2[...].astype(jnp.bfloat16), w2[...], preferred_element_type=jnp.float32)
         + bfr[...])
    out_ref[...] = jnp.tanh(p)


def kernel(input_seqs, input_lens, emb, Kg0, bg0, Kc0, bc0, Kg1, bg1, Kc1, bc1,
           Kg2, bg2, Kc2, bc2, W, b):
    f32 = jnp.float32
    bf16 = jnp.bfloat16

    kx0 = jnp.concatenate([Kg0[:EMB], Kc0[:EMB]], axis=1)
    e0 = jnp.zeros((OHV, 3 * S0), f32).at[:VOCAB].set(emb.astype(f32) @ kx0)
    e0 = e0.astype(bf16)
    gh0w = Kg0[EMB:].astype(bf16)
    ch0w = Kc0[EMB:].astype(bf16)
    x1w = jnp.concatenate([Kg1[:S0], Kc1[:S0]], axis=1).astype(bf16)
    gh1w = Kg1[S0:].astype(bf16)
    ch1w = Kc1[S0:].astype(bf16)
    x2w = jnp.concatenate([Kg2[:S1], Kc2[:S1]], axis=1).astype(bf16)
    gh2w = Kg2[S1:].astype(bf16)
    ch2w = Kc2[S1:].astype(bf16)
    wt = jnp.transpose(W)
    w0t = wt[:S0].astype(bf16)
    w1t = wt[S0:S0 + S1].astype(bf16)
    w2t = wt[S0 + S1:].astype(bf16)

    seqs = jnp.transpose(input_seqs).reshape(T, 1, B)
    lens = input_lens.reshape(B, 1)
    bg0r = bg0.reshape(1, -1)
    bc0r = bc0.reshape(1, -1)
    bg1r = bg1.reshape(1, -1)
    bc1r = bc1.reshape(1, -1)
    bg2r = bg2.reshape(1, -1)
    bc2r = bc2.reshape(1, -1)
    bfr = b.reshape(1, -1)

    full = lambda shape: pl.BlockSpec(shape, lambda i: tuple(0 for _ in shape))
    anyspec = pl.BlockSpec(memory_space=pl.ANY)

    out = pl.pallas_call(
        _gru_body,
        grid=(1,),
        in_specs=[
            pl.BlockSpec((T, 1, BH), lambda i: (0, 0, i)),
            pl.BlockSpec((BH, 1), lambda i: (i, 0)),
            full((1, 2 * S0)), full((1, S0)),
            full((1, 2 * S1)), full((1, S1)),
            full((1, 2 * S2)), full((1, S2)),
            full((1, LATENT)),
        ] + [anyspec] * 12,
        out_specs=pl.BlockSpec((BH, LATENT), lambda i: (i, 0)),
        out_shape=jax.ShapeDtypeStruct((B, LATENT), f32),
        scratch_shapes=[
            pltpu.VMEM((OHV, 3 * S0), bf16),
            pltpu.VMEM((S0, 2 * S0), bf16),
            pltpu.VMEM((S0, S0), bf16),
            pltpu.VMEM((S0, 3 * S1), bf16),
            pltpu.VMEM((S1, 2 * S1), bf16),
            pltpu.VMEM((S1, S1), bf16),
            pltpu.VMEM((S1, 3 * S2), bf16),
            pltpu.VMEM((S2, 2 * S2), bf16),
            pltpu.VMEM((S2, S2), bf16),
            pltpu.VMEM((S0, LATENT), bf16),
            pltpu.VMEM((S1, LATENT), bf16),
            pltpu.VMEM((S2, LATENT), bf16),
            pltpu.VMEM((BH, S0), f32),
            pltpu.VMEM((BH, S1), f32),
            pltpu.VMEM((BH, S2), f32),
            pltpu.SemaphoreType.DMA((12,)),
        ],
        compiler_params=pltpu.CompilerParams(
            dimension_semantics=("arbitrary",),
            vmem_limit_bytes=64 * 1024 * 1024,
        ),
        name="mini_cddd_gru",
    )(seqs, lens, bg0r, bc0r, bg1r, bc1r, bg2r, bc2r, bfr,
      e0, gh0w, ch0w, x1w, gh1w, ch1w, x2w, gh2w, ch2w, w0t, w1t, w2t)
    return out


# speedup vs baseline: 1.9482x; 1.0049x over previous
import jax
import jax.numpy as jnp
from jax.experimental import pallas as pl
from jax.experimental.pallas import tpu as pltpu

VOCAB = 40
EMB = 32
S0, S1, S2 = 512, 1024, 2048
LATENT = 512
B, T = 256, 128
BH = B
OHV = 128


def _gru_body(seq_ref, len_ref, bg0r, bc0r, bg1r, bc1r, bg2r, bc2r, bfr,
              e0a, gh0a, ch0a, x1a, gh1a, ch1a, x2a, gh2a, ch2a, w0a, w1a, w2a,
              out_ref,
              e0, gh0, ch0, x1, gh1, ch1, x2, gh2, ch2, w0, w1, w2,
              h0, h1, h2, sems):
    srcs = (e0a, gh0a, ch0a, x1a, gh1a, ch1a, x2a, gh2a, ch2a, w0a, w1a, w2a)
    dsts = (e0, gh0, ch0, x1, gh1, ch1, x2, gh2, ch2, w0, w1, w2)
    for i, (s, d) in enumerate(zip(srcs, dsts)):
        pltpu.make_async_copy(s, d, sems.at[i]).start()
    for i, (s, d) in enumerate(zip(srcs, dsts)):
        pltpu.make_async_copy(s, d, sems.at[i]).wait()

    h0[...] = jnp.zeros((BH, S0), jnp.float32)
    h1[...] = jnp.zeros((BH, S1), jnp.float32)
    h2[...] = jnp.zeros((BH, S2), jnp.float32)

    viota = jax.lax.broadcasted_iota(jnp.int32, (OHV, BH), 0)

    def cell(xc, h_ref, ghw, chw, bg, bc, outs):
        h = h_ref[...]
        hb = h.astype(jnp.bfloat16)
        grp = (xc[:, :outs]
               + jnp.dot(hb, ghw[:, :outs], preferred_element_type=jnp.float32)
               + bg[:, :outs])
        r = 0.5 * jnp.tanh(0.5 * grp) + 0.5
        rhb = (r * h).astype(jnp.bfloat16)
        gzp = (xc[:, outs:2 * outs]
               + jnp.dot(hb, ghw[:, outs:], preferred_element_type=jnp.float32)
               + bg[:, outs:])
        c = jnp.tanh(
            xc[:, 2 * outs:]
            + jnp.dot(rhb, chw[...], preferred_element_type=jnp.float32)
            + bc[...])
        z = 0.5 * jnp.tanh(0.5 * gzp) + 0.5
        return c + z * (h - c)

    def step(t, _):
        ids = seq_ref[t]
        ohT = jnp.where(ids == viota, 1.0, 0.0)
        oh = jnp.transpose(ohT).astype(jnp.bfloat16)
        xc0 = jnp.dot(oh, e0[...], preferred_element_type=jnp.float32)
        n0 = cell(xc0, h0, gh0, ch0, bg0r, bc0r, S0)
        xc1 = jnp.dot(n0.astype(jnp.bfloat16), x1[...], preferred_element_type=jnp.float32)
        n1 = cell(xc1, h1, gh1, ch1, bg1r, bc1r, S1)
        xc2 = jnp.dot(n1.astype(jnp.bfloat16), x2[...], preferred_element_type=jnp.float32)
        n2 = cell(xc2, h2, gh2, ch2, bg2r, bc2r, S2)
        m = len_ref[...] > t
        h0[...] = jnp.where(m, n0, h0[...])
        h1[...] = jnp.where(m, n1, h1[...])
        h2[...] = jnp.where(m, n2, h2[...])
        return 0

    jax.lax.fori_loop(0, T, step, 0)

    p = (jnp.dot(h0[...].astype(jnp.bfloat16), w0[...], preferred_element_type=jnp.float32)
         + jnp.dot(h1[...].astype(jnp.bfloat16), w1[...], preferred_element_type=jnp.float32)
         + jnp.dot(h2[...].astype(jnp.bfloat16), w2[...], preferred_element_type=jnp.float32)
         + bfr[...])
    out_ref[...] = jnp.tanh(p)


def kernel(input_seqs, input_lens, emb, Kg0, bg0, Kc0, bc0, Kg1, bg1, Kc1, bc1,
           Kg2, bg2, Kc2, bc2, W, b):
    f32 = jnp.float32
    bf16 = jnp.bfloat16

    kx0 = jnp.concatenate([Kg0[:EMB], Kc0[:EMB]], axis=1)
    e0 = jnp.zeros((OHV, 3 * S0), f32).at[:VOCAB].set(emb.astype(f32) @ kx0)
    e0 = e0.astype(bf16)
    gh0w = Kg0[EMB:].astype(bf16)
    ch0w = Kc0[EMB:].astype(bf16)
    x1w = jnp.concatenate([Kg1[:S0], Kc1[:S0]], axis=1).astype(bf16)
    gh1w = Kg1[S0:].astype(bf16)
    ch1w = Kc1[S0:].astype(bf16)
    x2w = jnp.concatenate([Kg2[:S1], Kc2[:S1]], axis=1).astype(bf16)
    gh2w = Kg2[S1:].astype(bf16)
    ch2w = Kc2[S1:].astype(bf16)
    wt = jnp.transpose(W)
    w0t = wt[:S0].astype(bf16)
    w1t = wt[S0:S0 + S1].astype(bf16)
    w2t = wt[S0 + S1:].astype(bf16)

    seqs = jnp.transpose(input_seqs).reshape(T, 1, B)
    lens = input_lens.reshape(B, 1)
    bg0r = bg0.reshape(1, -1)
    bc0r = bc0.reshape(1, -1)
    bg1r = bg1.reshape(1, -1)
    bc1r = bc1.reshape(1, -1)
    bg2r = bg2.reshape(1, -1)
    bc2r = bc2.reshape(1, -1)
    bfr = b.reshape(1, -1)

    full = lambda shape: pl.BlockSpec(shape, lambda i: tuple(0 for _ in shape))
    anyspec = pl.BlockSpec(memory_space=pl.ANY)

    out = pl.pallas_call(
        _gru_body,
        grid=(1,),
        in_specs=[
            pl.BlockSpec((T, 1, BH), lambda i: (0, 0, i)),
            pl.BlockSpec((BH, 1), lambda i: (i, 0)),
            full((1, 2 * S0)), full((1, S0)),
            full((1, 2 * S1)), full((1, S1)),
            full((1, 2 * S2)), full((1, S2)),
            full((1, LATENT)),
        ] + [anyspec] * 12,
        out_specs=pl.BlockSpec((BH, LATENT), lambda i: (i, 0)),
        out_shape=jax.ShapeDtypeStruct((B, LATENT), f32),
        scratch_shapes=[
            pltpu.VMEM((OHV, 3 * S0), bf16),
            pltpu.VMEM((S0, 2 * S0), bf16),
            pltpu.VMEM((S0, S0), bf16),
            pltpu.VMEM((S0, 3 * S1), bf16),
            pltpu.VMEM((S1, 2 * S1), bf16),
            pltpu.VMEM((S1, S1), bf16),
            pltpu.VMEM((S1, 3 * S2), bf16),
            pltpu.VMEM((S2, 2 * S2), bf16),
            pltpu.VMEM((S2, S2), bf16),
            pltpu.VMEM((S0, LATENT), bf16),
            pltpu.VMEM((S1, LATENT), bf16),
            pltpu.VMEM((S2, LATENT), bf16),
            pltpu.VMEM((BH, S0), f32),
            pltpu.VMEM((BH, S1), f32),
            pltpu.VMEM((BH, S2), f32),
            pltpu.SemaphoreType.DMA((12,)),
        ],
        compiler_params=pltpu.CompilerParams(
            dimension_semantics=("arbitrary",),
            vmem_limit_bytes=64 * 1024 * 1024,
        ),
        name="mini_cddd_gru",
    )(seqs, lens, bg0r, bc0r, bg1r, bc1r, bg2r, bc2r, bfr,
      e0, gh0w, ch0w, x1w, gh1w, ch1w, x2w, gh2w, ch2w, w0t, w1t, w2t)
    return out


# speedup vs baseline: 2.0307x; 1.0424x over previous
import jax
import jax.numpy as jnp
from jax.experimental import pallas as pl
from jax.experimental.pallas import tpu as pltpu

VOCAB = 40
EMB = 32
S0, S1, S2 = 512, 1024, 2048
LATENT = 512
B, T = 256, 128
OHV = 128


def _gru_body(seq_ref, len_ref, bg0r, bc0r, bg1r, bc1r, bg2r, bc2r, bfr,
              g0a, c0a, g1a, c1a, g2a, c2a, w0a, w1a, w2a,
              out_ref,
              g0w, c0w, g1w, c1w, g2w, c2w, w0, w1, w2,
              xh0, xh1, xh2, h0, h1, h2, sems):
    f32 = jnp.float32
    bf16 = jnp.bfloat16

    srcs = (g0a, c0a, g1a, c1a, g2a, c2a, w0a, w1a, w2a)
    dsts = (g0w, c0w, g1w, c1w, g2w, c2w, w0, w1, w2)
    for i, (s, d) in enumerate(zip(srcs, dsts)):
        pltpu.make_async_copy(s, d, sems.at[i]).start()
    for i, (s, d) in enumerate(zip(srcs, dsts)):
        pltpu.make_async_copy(s, d, sems.at[i]).wait()

    h0[...] = jnp.zeros((B, S0), f32)
    h1[...] = jnp.zeros((B, S1), f32)
    h2[...] = jnp.zeros((B, S2), f32)

    viota = jax.lax.broadcasted_iota(jnp.int32, (OHV, B), 0)

    def cell(xh, ins, h_ref, gw, cw, bg, bc, outs, xb):
        xh[:, :ins] = xb
        h = h_ref[...]
        xh[:, ins:] = h.astype(bf16)
        gp = jnp.dot(xh[...], gw[...], preferred_element_type=f32) + bg[...]
        r = 0.5 * jnp.tanh(0.5 * gp[:, :outs]) + 0.5
        z = 0.5 * jnp.tanh(0.5 * gp[:, outs:]) + 0.5
        xh[:, ins:] = (r * h).astype(bf16)
        c = jnp.tanh(jnp.dot(xh[...], cw[...], preferred_element_type=f32) + bc[...])
        return c + z * (h - c)

    def step(t, _):
        ids = seq_ref[t]
        ohT = jnp.where(ids == viota, 1.0, 0.0)
        oh = jnp.transpose(ohT).astype(bf16)
        n0 = cell(xh0, OHV, h0, g0w, c0w, bg0r, bc0r, S0, oh)
        n1 = cell(xh1, S0, h1, g1w, c1w, bg1r, bc1r, S1, n0.astype(bf16))
        n2 = cell(xh2, S1, h2, g2w, c2w, bg2r, bc2r, S2, n1.astype(bf16))
        m = len_ref[...] > t
        h0[...] = jnp.where(m, n0, h0[...])
        h1[...] = jnp.where(m, n1, h1[...])
        h2[...] = jnp.where(m, n2, h2[...])
        return 0

    jax.lax.fori_loop(0, T, step, 0)

    p = (jnp.dot(h0[...].astype(bf16), w0[...], preferred_element_type=f32)
         + jnp.dot(h1[...].astype(bf16), w1[...], preferred_element_type=f32)
         + jnp.dot(h2[...].astype(bf16), w2[...], preferred_element_type=f32)
         + bfr[...])
    out_ref[...] = jnp.tanh(p)


def kernel(input_seqs, input_lens, emb, Kg0, bg0, Kc0, bc0, Kg1, bg1, Kc1, bc1,
           Kg2, bg2, Kc2, bc2, W, b):
    f32 = jnp.float32
    bf16 = jnp.bfloat16

    def l0(K):
        top = jnp.zeros((OHV, K.shape[1]), f32).at[:VOCAB].set(emb.astype(f32) @ K[:EMB])
        return jnp.concatenate([top, K[EMB:]], axis=0).astype(bf16)

    g0w = l0(Kg0)
    c0w = l0(Kc0)
    g1w = Kg1.astype(bf16)
    c1w = Kc1.astype(bf16)
    g2w = Kg2.astype(bf16)
    c2w = Kc2.astype(bf16)
    wt = jnp.transpose(W)
    w0t = wt[:S0].astype(bf16)
    w1t = wt[S0:S0 + S1].astype(bf16)
    w2t = wt[S0 + S1:].astype(bf16)

    seqs = jnp.transpose(input_seqs).reshape(T, 1, B)
    lens = input_lens.reshape(B, 1)
    bg0r = bg0.reshape(1, -1)
    bc0r = bc0.reshape(1, -1)
    bg1r = bg1.reshape(1, -1)
    bc1r = bc1.reshape(1, -1)
    bg2r = bg2.reshape(1, -1)
    bc2r = bc2.reshape(1, -1)
    bfr = b.reshape(1, -1)

    full = lambda shape: pl.BlockSpec(shape, lambda i: tuple(0 for _ in shape))
    anyspec = pl.BlockSpec(memory_space=pl.ANY)

    out = pl.pallas_call(
        _gru_body,
        grid=(1,),
        in_specs=[
            pl.BlockSpec((T, 1, B), lambda i: (0, 0, 0)),
            pl.BlockSpec((B, 1), lambda i: (0, 0)),
            full((1, 2 * S0)), full((1, S0)),
            full((1, 2 * S1)), full((1, S1)),
            full((1, 2 * S2)), full((1, S2)),
            full((1, LATENT)),
        ] + [anyspec] * 9,
        out_specs=pl.BlockSpec((B, LATENT), lambda i: (0, 0)),
        out_shape=jax.ShapeDtypeStruct((B, LATENT), f32),
        scratch_shapes=[
            pltpu.VMEM((OHV + S0, 2 * S0), bf16),
            pltpu.VMEM((OHV + S0, S0), bf16),
            pltpu.VMEM((S0 + S1, 2 * S1), bf16),
            pltpu.VMEM((S0 + S1, S1), bf16),
            pltpu.VMEM((S1 + S2, 2 * S2), bf16),
            pltpu.VMEM((S1 + S2, S2), bf16),
            pltpu.VMEM((S0, LATENT), bf16),
            pltpu.VMEM((S1, LATENT), bf16),
            pltpu.VMEM((S2, LATENT), bf16),
            pltpu.VMEM((B, OHV + S0), bf16),
            pltpu.VMEM((B, S0 + S1), bf16),
            pltpu.VMEM((B, S1 + S2), bf16),
            pltpu.VMEM((B, S0), f32),
            pltpu.VMEM((B, S1), f32),
            pltpu.VMEM((B, S2), f32),
            pltpu.SemaphoreType.DMA((9,)),
        ],
        compiler_params=pltpu.CompilerParams(
            dimension_semantics=("arbitrary",),
            vmem_limit_bytes=64 * 1024 * 1024,
        ),
        name="mini_cddd_gru",
    )(seqs, lens, bg0r, bc0r, bg1r, bc1r, bg2r, bc2r, bfr,
      g0w, c0w, g1w, c1w, g2w, c2w, w0t, w1t, w2t)
    return out
